# baseline (speedup 1.0000x reference)
import jax
import jax.numpy as jnp
from jax import lax
from jax.experimental import pallas as pl
from jax.experimental.pallas import tpu as pltpu

_BS = 8
_BL = 128
_BT = _BS * _BL


def _fused_kernel(x_ref, w1_ref, b1_ref, w2_ref, b2_ref,
                  w1k_ref, b1k_ref, w2k_ref, b2k_ref,
                  o_ref, p1_ref, h_ref):

    def conv1_row(py, carry):
        for co in range(8):
            acc = None
            for di in range(3):
                rows = x_ref[0, pl.ds(2 * py + di, 2)]
                for dj in range(3):
                    t = rows[:, dj:dj + 26] * w1_ref[co * 9 + di * 3 + dj]
                    acc = t if acc is None else acc + t
            m = jnp.maximum(acc[0], acc[1])
            m = m.reshape(13, 2, _BS, _BL)
            m = jnp.maximum(m[:, 0], m[:, 1])
            p1_ref[co, py] = jnp.maximum(m + b1_ref[co], 0.0)
        return carry

    lax.fori_loop(0, 13, conv1_row, 0, unroll=2)

    def conv2_chan(co, carry):
        acc = jnp.zeros((10, 10, _BS, _BL), jnp.float32)
        for ci in range(8):
            for di in range(3):
                for dj in range(3):
                    w = w2_ref[(co * 8 + ci) * 9 + di * 3 + dj]
                    acc = acc + p1_ref[ci, di:di + 10, dj:dj + 10] * w
        m = acc.reshape(5, 2, 10, _BS, _BL)
        m = jnp.maximum(m[:, 0], m[:, 1])
        m = m.reshape(5, 5, 2, _BS, _BL)
        m = jnp.maximum(m[:, :, 0], m[:, :, 1])
        m = jnp.maximum(m + b2_ref[co], 0.0)
        xf = m.reshape(200, _BL).astype(jnp.bfloat16)
        part = jax.lax.dot(w1k_ref[co], xf,
                           preferred_element_type=jnp.float32)

        @pl.when(co == 0)
        def _():
            h_ref[...] = part

        @pl.when(co != 0)
        def _():
            h_ref[...] = h_ref[...] + part

        return carry

    lax.fori_loop(0, 8, conv2_chan, 0, unroll=2)

    h = jnp.maximum(h_ref[...] + b1k_ref[...], 0.0)
    logits = jax.lax.dot(w2k_ref[...], h.astype(jnp.bfloat16),
                         preferred_element_type=jnp.float32)
    logits = (logits + b2k_ref[...]).reshape(10, _BS, _BL)
    mx = jnp.max(logits, axis=0)
    lse = jnp.log(jnp.sum(jnp.exp(logits - mx), axis=0))
    o_ref[0] = logits - (mx + lse)


@jax.jit
def _forward(x, w1, b1, w2, b2, wf1, bf1, wf2, bf2):
    f32 = jnp.float32
    bf16 = jnp.bfloat16
    N = x.shape[0]
    G = -(-N // _BT)
    B_pad = G * _BT

    x2 = x.reshape(N, 28, 28).astype(f32)
    if B_pad != N:
        x2 = jnp.pad(x2, ((0, B_pad - N), (0, 0), (0, 0)))
    xk = x2.reshape(G, _BS, _BL, 28, 28).transpose(0, 3, 4, 1, 2)

    w1f = w1.reshape(-1).astype(f32)
    w2f = w2.reshape(-1).astype(f32)
    b1f = b1.astype(f32)
    b2f = b2.astype(f32)

    eye = jnp.eye(_BS, dtype=f32)
    w1k = jnp.kron(wf1.astype(f32), eye).astype(bf16)
    w1k = w1k.reshape(512, 8, 200).transpose(1, 0, 2)
    w2k = jnp.kron(wf2.astype(f32), eye).astype(bf16)
    b1k = jnp.repeat(bf1.astype(f32), _BS).reshape(-1, 1)
    b2k = jnp.repeat(bf2.astype(f32), _BS).reshape(-1, 1)

    smem = pl.BlockSpec(memory_space=pltpu.MemorySpace.SMEM)
    single = pl.Buffered(1)
    out = pl.pallas_call(
        _fused_kernel,
        grid=(G,),
        in_specs=[
            pl.BlockSpec((1, 28, 28, _BS, _BL), lambda g: (g, 0, 0, 0, 0)),
            smem, smem, smem, smem,
            pl.BlockSpec((8, 64 * _BS, 200), lambda g: (0, 0, 0),
                         pipeline_mode=single),
            pl.BlockSpec((64 * _BS, 1), lambda g: (0, 0),
                         pipeline_mode=single),
            pl.BlockSpec((10 * _BS, 64 * _BS), lambda g: (0, 0),
                         pipeline_mode=single),
            pl.BlockSpec((10 * _BS, 1), lambda g: (0, 0),
                         pipeline_mode=single),
        ],
        out_specs=pl.BlockSpec((1, 10, _BS, _BL), lambda g: (g, 0, 0, 0)),
        out_shape=jax.ShapeDtypeStruct((G, 10, _BS, _BL), f32),
        scratch_shapes=[
            pltpu.VMEM((8, 13, 13, _BS, _BL), f32),
            pltpu.VMEM((64 * _BS, _BL), f32),
        ],
        compiler_params=pltpu.CompilerParams(
            dimension_semantics=("parallel",),
            vmem_limit_bytes=40 * 1024 * 1024,
        ),
    )(xk, w1f, b1f, w2f, b2f, w1k, b1k, w2k, b2k)

    return out.transpose(0, 2, 3, 1).reshape(B_pad, 10)[:N]


def kernel(x, w1, b1, w2, b2, wf1, bf1, wf2, bf2):
    return _forward(x, w1, b1, w2, b2, wf1, bf1, wf2, bf2)


# speedup vs baseline: 1.5562x; 1.5562x over previous
import jax
import jax.numpy as jnp
from jax import lax
from jax.experimental import pallas as pl
from jax.experimental.pallas import tpu as pltpu

_BS = 8
_BL = 128
_BT = _BS * _BL


def _fused_kernel(x_ref, w1_ref, b1_ref, w2_ref, b2_ref,
                  w1k_ref, b1k_ref, w2k_ref, b2k_ref,
                  o_ref, p1_ref, fc_ref):

    def conv1_row(py, carry):
        for co in range(8):
            acc = None
            for di in range(3):
                rows = x_ref[0, pl.ds(2 * py + di, 2)]
                for dj in range(3):
                    t = rows[:, dj:dj + 26] * w1_ref[co * 9 + di * 3 + dj]
                    acc = t if acc is None else acc + t
            m = jnp.maximum(acc[0], acc[1])
            m = m.reshape(13, 2, _BS, _BL)
            m = jnp.maximum(m[:, 0], m[:, 1])
            p1_ref[co, py] = jnp.maximum(m + b1_ref[co], 0.0)
        return carry

    lax.fori_loop(0, 13, conv1_row, 0, unroll=2)

    def conv2_chan(co, carry):
        acc = jnp.zeros((10, 10, _BS, _BL), jnp.float32)
        for ci in range(8):
            for di in range(3):
                for dj in range(3):
                    w = w2_ref[(co * 8 + ci) * 9 + di * 3 + dj]
                    acc = acc + p1_ref[ci, di:di + 10, dj:dj + 10] * w
        m = acc.reshape(5, 2, 10, _BS, _BL)
        m = jnp.maximum(m[:, 0], m[:, 1])
        m = m.reshape(5, 5, 2, _BS, _BL)
        m = jnp.maximum(m[:, :, 0], m[:, :, 1])
        m = jnp.maximum(m + b2_ref[co], 0.0)
        base = pl.multiple_of(co * 200, 8)
        fc_ref[pl.ds(base, 200)] = m.reshape(200, _BL)
        return carry

    lax.fori_loop(0, 8, conv2_chan, 0, unroll=2)

    xf = fc_ref[...].astype(jnp.bfloat16)
    h = jax.lax.dot(w1k_ref[...], xf,
                    preferred_element_type=jnp.float32)
    h = jnp.maximum(h + b1k_ref[...], 0.0)
    logits = jax.lax.dot(w2k_ref[...], h.astype(jnp.bfloat16),
                         preferred_element_type=jnp.float32)
    logits = (logits + b2k_ref[...]).reshape(10, _BS, _BL)
    mx = jnp.max(logits, axis=0)
    lse = jnp.log(jnp.sum(jnp.exp(logits - mx), axis=0))
    o_ref[0] = logits - (mx + lse)


@jax.jit
def _forward(x, w1, b1, w2, b2, wf1, bf1, wf2, bf2):
    f32 = jnp.float32
    bf16 = jnp.bfloat16
    N = x.shape[0]
    G = -(-N // _BT)
    B_pad = G * _BT

    x2 = x.reshape(N, 28, 28).astype(f32)
    if B_pad != N:
        x2 = jnp.pad(x2, ((0, B_pad - N), (0, 0), (0, 0)))
    xk = x2.reshape(G, _BS, _BL, 28, 28).transpose(0, 3, 4, 1, 2)

    w1f = w1.reshape(-1).astype(f32)
    w2f = w2.reshape(-1).astype(f32)
    b1f = b1.astype(f32)
    b2f = b2.astype(f32)

    eye = jnp.eye(_BS, dtype=f32)
    w1k = jnp.kron(wf1.astype(f32), eye).astype(bf16)
    w2k = jnp.kron(wf2.astype(f32), eye).astype(bf16)
    b1k = jnp.repeat(bf1.astype(f32), _BS).reshape(-1, 1)
    b2k = jnp.repeat(bf2.astype(f32), _BS).reshape(-1, 1)

    smem = pl.BlockSpec(memory_space=pltpu.MemorySpace.SMEM)
    single = pl.Buffered(1)
    out = pl.pallas_call(
        _fused_kernel,
        grid=(G,),
        in_specs=[
            pl.BlockSpec((1, 28, 28, _BS, _BL), lambda g: (g, 0, 0, 0, 0)),
            smem, smem, smem, smem,
            pl.BlockSpec((64 * _BS, 200 * _BS), lambda g: (0, 0),
                         pipeline_mode=single),
            pl.BlockSpec((64 * _BS, 1), lambda g: (0, 0),
                         pipeline_mode=single),
            pl.BlockSpec((10 * _BS, 64 * _BS), lambda g: (0, 0),
                         pipeline_mode=single),
            pl.BlockSpec((10 * _BS, 1), lambda g: (0, 0),
                         pipeline_mode=single),
        ],
        out_specs=pl.BlockSpec((1, 10, _BS, _BL), lambda g: (g, 0, 0, 0)),
        out_shape=jax.ShapeDtypeStruct((G, 10, _BS, _BL), f32),
        scratch_shapes=[
            pltpu.VMEM((8, 13, 13, _BS, _BL), f32),
            pltpu.VMEM((200 * _BS, _BL), f32),
        ],
        compiler_params=pltpu.CompilerParams(
            dimension_semantics=("parallel",),
            vmem_limit_bytes=40 * 1024 * 1024,
        ),
    )(xk, w1f, b1f, w2f, b2f, w1k, b1k, w2k, b2k)

    return out.transpose(0, 2, 3, 1).reshape(B_pad, 10)[:N]


def kernel(x, w1, b1, w2, b2, wf1, bf1, wf2, bf2):
    return _forward(x, w1, b1, w2, b2, wf1, bf1, wf2, bf2)


# speedup vs baseline: 1.5578x; 1.0010x over previous
import jax
import jax.numpy as jnp
from jax import lax
from jax.experimental import pallas as pl
from jax.experimental.pallas import tpu as pltpu

_BS = 8
_BL = 128
_BT = _BS * _BL


def _fused_kernel(x_ref, w1_ref, b1_ref, w2_ref, b2_ref,
                  w1k_ref, b1k_ref, w2k_ref, b2k_ref,
                  o_ref, p1_ref, fc_ref):

    def conv1_row(py, carry):
        for co in range(8):
            acc = None
            for di in range(3):
                rows = x_ref[0, pl.ds(2 * py + di, 2)]
                for dj in range(3):
                    t = rows[:, dj:dj + 26] * w1_ref[co * 9 + di * 3 + dj]
                    acc = t if acc is None else acc + t
            m = jnp.maximum(acc[0], acc[1])
            m = m.reshape(13, 2, _BS, _BL)
            m = jnp.maximum(m[:, 0], m[:, 1])
            p1_ref[co, py] = jnp.maximum(m + b1_ref[co], 0.0)
        return carry

    lax.fori_loop(0, 13, conv1_row, 0, unroll=4)

    def conv2_chan(co, carry):
        acc = jnp.zeros((10, 10, _BS, _BL), jnp.float32)
        for ci in range(8):
            for di in range(3):
                for dj in range(3):
                    w = w2_ref[(co * 8 + ci) * 9 + di * 3 + dj]
                    acc = acc + p1_ref[ci, di:di + 10, dj:dj + 10] * w
        m = acc.reshape(5, 2, 10, _BS, _BL)
        m = jnp.maximum(m[:, 0], m[:, 1])
        m = m.reshape(5, 5, 2, _BS, _BL)
        m = jnp.maximum(m[:, :, 0], m[:, :, 1])
        m = jnp.maximum(m + b2_ref[co], 0.0)
        base = pl.multiple_of(co * 200, 8)
        fc_ref[pl.ds(base, 200)] = m.reshape(200, _BL)
        return carry

    lax.fori_loop(0, 8, conv2_chan, 0, unroll=2)

    xf = fc_ref[...].astype(jnp.bfloat16)
    h = jax.lax.dot(w1k_ref[...], xf,
                    preferred_element_type=jnp.float32)
    h = jnp.maximum(h + b1k_ref[...], 0.0)
    logits = jax.lax.dot(w2k_ref[...], h.astype(jnp.bfloat16),
                         preferred_element_type=jnp.float32)
    logits = (logits + b2k_ref[...]).reshape(10, _BS, _BL)
    mx = jnp.max(logits, axis=0)
    lse = jnp.log(jnp.sum(jnp.exp(logits - mx), axis=0))
    o_ref[0] = logits - (mx + lse)


@jax.jit
def _forward(x, w1, b1, w2, b2, wf1, bf1, wf2, bf2):
    f32 = jnp.float32
    bf16 = jnp.bfloat16
    N = x.shape[0]
    G = -(-N // _BT)
    B_pad = G * _BT

    x2 = x.reshape(N, 28, 28).astype(f32)
    if B_pad != N:
        x2 = jnp.pad(x2, ((0, B_pad - N), (0, 0), (0, 0)))
    xk = x2.reshape(G, _BS, _BL, 28, 28).transpose(0, 3, 4, 1, 2)

    w1f = w1.reshape(-1).astype(f32)
    w2f = w2.reshape(-1).astype(f32)
    b1f = b1.astype(f32)
    b2f = b2.astype(f32)

    eye = jnp.eye(_BS, dtype=f32)
    w1k = jnp.kron(wf1.astype(f32), eye).astype(bf16)
    w2k = jnp.kron(wf2.astype(f32), eye).astype(bf16)
    b1k = jnp.repeat(bf1.astype(f32), _BS).reshape(-1, 1)
    b2k = jnp.repeat(bf2.astype(f32), _BS).reshape(-1, 1)

    smem = pl.BlockSpec(memory_space=pltpu.MemorySpace.SMEM)
    single = pl.Buffered(1)
    out = pl.pallas_call(
        _fused_kernel,
        grid=(G,),
        in_specs=[
            pl.BlockSpec((1, 28, 28, _BS, _BL), lambda g: (g, 0, 0, 0, 0)),
            smem, smem, smem, smem,
            pl.BlockSpec((64 * _BS, 200 * _BS), lambda g: (0, 0),
                         pipeline_mode=single),
            pl.BlockSpec((64 * _BS, 1), lambda g: (0, 0),
                         pipeline_mode=single),
            pl.BlockSpec((10 * _BS, 64 * _BS), lambda g: (0, 0),
                         pipeline_mode=single),
            pl.BlockSpec((10 * _BS, 1), lambda g: (0, 0),
                         pipeline_mode=single),
        ],
        out_specs=pl.BlockSpec((1, 10, _BS, _BL), lambda g: (g, 0, 0, 0)),
        out_shape=jax.ShapeDtypeStruct((G, 10, _BS, _BL), f32),
        scratch_shapes=[
            pltpu.VMEM((8, 13, 13, _BS, _BL), f32),
            pltpu.VMEM((200 * _BS, _BL), f32),
        ],
        compiler_params=pltpu.CompilerParams(
            dimension_semantics=("parallel",),
            vmem_limit_bytes=40 * 1024 * 1024,
        ),
    )(xk, w1f, b1f, w2f, b2f, w1k, b1k, w2k, b2k)

    return out.transpose(0, 2, 3, 1).reshape(B_pad, 10)[:N]


def kernel(x, w1, b1, w2, b2, wf1, bf1, wf2, bf2):
    return _forward(x, w1, b1, w2, b2, wf1, bf1, wf2, bf2)
